# baseline (speedup 1.0000x reference)
import functools

import jax
import jax.numpy as jnp
from jax import lax
from jax.experimental import pallas as pl
from jax.experimental.pallas import tpu as pltpu
from jax.experimental.pallas import tpu_sc as plsc

_R = 8


def kernel(x, emb):
    batch, seq_len, model_dim = x.shape
    info = plsc.get_sparse_core_info()
    nc, ns = info.num_cores, info.num_subcores
    nw = nc * ns
    lanes = info.num_lanes
    rows_pw = seq_len // nw
    n_chunks = rows_pw // _R
    cr = _R * model_dim // lanes
    x3 = x.reshape(-1, lanes)
    emb3 = emb.reshape(-1, lanes)
    b_stride = seq_len * model_dim // lanes
    mesh = plsc.VectorSubcoreMesh(core_axis_name="c", subcore_axis_name="s")

    @functools.partial(
        pl.kernel,
        out_type=jax.ShapeDtypeStruct(x3.shape, x.dtype),
        mesh=mesh,
        scratch_types=[
            pltpu.VMEM((cr, lanes), jnp.float32),
            pltpu.VMEM((cr, lanes), jnp.float32),
        ],
    )
    def sc_add(x_hbm, emb_hbm, out_hbm, embbuf, xbuf):
        cid = lax.axis_index("c")
        sid = lax.axis_index("s")
        wid = sid * nc + cid

        def chunk_body(g, carry):
            e0 = (wid * n_chunks + g) * cr
            pltpu.sync_copy(emb_hbm.at[pl.ds(e0, cr), :], embbuf)
            for b in range(batch):
                r0 = b * b_stride + e0
                pltpu.sync_copy(x_hbm.at[pl.ds(r0, cr), :], xbuf)

                def add_body(i, c):
                    plsc.addupdate(xbuf.at[i], embbuf[i])
                    return c

                lax.fori_loop(0, cr, add_body, 0)

                pltpu.sync_copy(xbuf, out_hbm.at[pl.ds(r0, cr), :])
            return carry

        lax.fori_loop(0, n_chunks, chunk_body, 0)

    out3 = sc_add(x3, emb3)
    return out3.reshape(batch, seq_len, model_dim)


# speedup vs baseline: 4.3106x; 4.3106x over previous
import functools

import jax
import jax.numpy as jnp
from jax import lax
from jax.experimental import pallas as pl
from jax.experimental.pallas import tpu as pltpu
from jax.experimental.pallas import tpu_sc as plsc

_R = 8
_W = 128


def kernel(x, emb):
    batch, seq_len, model_dim = x.shape
    info = plsc.get_sparse_core_info()
    nc, ns = info.num_cores, info.num_subcores
    nw = nc * ns
    rows_pw = seq_len // nw
    n_chunks = rows_pw // _R
    cr = _R * model_dim // _W
    x4 = x.reshape(-1, _W)
    emb4 = emb.reshape(-1, _W)
    b_stride = seq_len * model_dim // _W
    mesh = plsc.VectorSubcoreMesh(core_axis_name="c", subcore_axis_name="s")

    @functools.partial(
        pl.kernel,
        out_type=jax.ShapeDtypeStruct(x4.shape, x.dtype),
        mesh=mesh,
        scratch_types=[
            pltpu.VMEM((cr, _W), jnp.float32),
            pltpu.VMEM((cr,), jnp.int32),
        ],
    )
    def sc_add(x_hbm, emb_hbm, out_hbm, xbuf, idxv):
        cid = lax.axis_index("c")
        sid = lax.axis_index("s")
        wid = sid * nc + cid

        def chunk_body(g, carry):
            e0 = (wid * n_chunks + g) * cr
            for j in range(cr // 16):
                idxv[pl.ds(j * 16, 16)] = lax.iota(jnp.int32, 16) + (e0 + j * 16)
            for b in range(batch):
                r0 = b * b_stride + e0
                pltpu.sync_copy(x_hbm.at[pl.ds(r0, cr), :], xbuf)
                pltpu.sync_copy(emb_hbm.at[idxv], xbuf, add=True)
                pltpu.sync_copy(xbuf, out_hbm.at[pl.ds(r0, cr), :])
            return carry

        lax.fori_loop(0, n_chunks, chunk_body, 0)

    out4 = sc_add(x4, emb4)
    return out4.reshape(batch, seq_len, model_dim)


# speedup vs baseline: 5.5698x; 1.2921x over previous
import functools

import jax
import jax.numpy as jnp
from jax import lax
from jax.experimental import pallas as pl
from jax.experimental.pallas import tpu as pltpu
from jax.experimental.pallas import tpu_sc as plsc

_R = 8
_W = 128
_NBUF = 3


def kernel(x, emb):
    batch, seq_len, model_dim = x.shape
    info = plsc.get_sparse_core_info()
    nc, ns = info.num_cores, info.num_subcores
    nw = nc * ns
    rows_pw = seq_len // nw
    n_chunks = rows_pw // _R
    cr = _R * model_dim // _W
    x4 = x.reshape(-1, _W)
    emb4 = emb.reshape(-1, _W)
    b_stride = seq_len * model_dim // _W
    mesh = plsc.VectorSubcoreMesh(core_axis_name="c", subcore_axis_name="s")

    @functools.partial(
        pl.kernel,
        out_type=jax.ShapeDtypeStruct(x4.shape, x.dtype),
        mesh=mesh,
        scratch_types=[
            [pltpu.VMEM((cr, _W), jnp.float32) for _ in range(_NBUF)],
            [pltpu.VMEM((cr,), jnp.int32) for _ in range(2)],
            [pltpu.SemaphoreType.DMA for _ in range(2 * _NBUF)],
        ],
    )
    def sc_add(x_hbm, emb_hbm, out_hbm, xbufs, idxvs, sems):
        cid = lax.axis_index("c")
        sid = lax.axis_index("s")
        wid = sid * nc + cid
        in_sems, out_sems = sems[:_NBUF], sems[_NBUF:]

        stages = [(g, b) for g in range(n_chunks) for b in range(batch)]

        def write_idx(g):
            iv = idxvs[g % 2]
            for j in range(cr // 16):
                iv[pl.ds(j * 16, 16)] = lax.iota(jnp.int32, 16) + (
                    (wid * n_chunks + g) * cr + j * 16
                )

        def start_in(k):
            g, b = stages[k]
            r0 = b * b_stride + (wid * n_chunks + g) * cr
            return pltpu.async_copy(
                x_hbm.at[pl.ds(r0, cr), :], xbufs[k % _NBUF], in_sems[k % _NBUF]
            )

        in_descs = [None] * len(stages)
        out_descs = [None] * len(stages)
        write_idx(0)
        in_descs[0] = start_in(0)
        for k, (g, b) in enumerate(stages):
            buf = k % _NBUF
            if b == 0 and g > 0:
                write_idx(g)
            if k + 1 < len(stages):
                if out_descs[k + 1 - _NBUF] is not None:
                    out_descs[k + 1 - _NBUF].wait()
                in_descs[k + 1] = start_in(k + 1)
            in_descs[k].wait()
            pltpu.sync_copy(emb_hbm.at[idxvs[g % 2]], xbufs[buf], add=True)
            r0 = b * b_stride + (wid * n_chunks + g) * cr
            out_descs[k] = pltpu.async_copy(
                xbufs[buf], out_hbm.at[pl.ds(r0, cr), :], out_sems[buf]
            )
        for k in range(len(stages) - _NBUF, len(stages)):
            out_descs[k].wait()

    out4 = sc_add(x4, emb4)
    return out4.reshape(batch, seq_len, model_dim)
